# baseline (speedup 1.0000x reference)
import functools

import jax
import jax.numpy as jnp
from jax import lax
from jax.experimental import pallas as pl
from jax.experimental.pallas import tpu as pltpu
from jax.experimental.pallas import tpu_sc as plsc

N_NODES = 10000
N_EDGES = 160000
NODE_VOCAB = 512
EDGE_VOCAB = 64
D_NODE_EMB = 16
D_EDGE_EMB = 32
H = 32
EDGE_HIDDEN = 128
N_STEPS = 6

NC = 2
NS = 16
NW = NC * NS
CH = 128
NCHUNK = 40
E_PAD = NW * NCHUNK * CH
EW = NCHUNK * CH
N_PAD = 10240
NODE_W = N_PAD // NW
STRIPE = N_PAD // NS
DUMP_ROW = 10200
BLK = 512

_f32 = jnp.float32
_i32 = jnp.int32


def _vocab_body(nemb, pw, pb, eemb, w1, b1, w2, b2, h0v, ewv):
    h0 = jnp.dot(nemb[...], pw[...], preferred_element_type=_f32) + pb[...]
    h0v[...] = jnp.maximum(h0, 0.0)
    z = jnp.dot(eemb[...], w1[...], preferred_element_type=_f32) + b1[...]
    z = jnp.maximum(z, 0.0)
    ewv[...] = jnp.dot(z, w2[...], preferred_element_type=_f32) + b2[...]


_vocab_call = pl.pallas_call(
    _vocab_body,
    out_shape=(
        jax.ShapeDtypeStruct((NODE_VOCAB, H), _f32),
        jax.ShapeDtypeStruct((EDGE_VOCAB, H * H), _f32),
    ),
)


def _nblk(b):
    return (b, 0)


def _rep(b):
    return (0, 0)


NG = (EDGE_VOCAB * H) // 128


def _store_q(q_out, hq):
    for g in range(NG):
        q_out[g] = hq[:, g * 128:(g + 1) * 128]


def _mm_body(a, b, o):
    _store_q(o, jnp.dot(a[...], b[...], preferred_element_type=_f32))


_q_call = pl.pallas_call(
    _mm_body,
    grid=(N_PAD // BLK,),
    in_specs=[
        pl.BlockSpec((BLK, H), _nblk),
        pl.BlockSpec((H, EDGE_VOCAB * H), _rep),
    ],
    out_specs=pl.BlockSpec((NG, BLK, 128), lambda b: (0, b, 0)),
    out_shape=jax.ShapeDtypeStruct((NG, N_PAD, 128), _f32),
)


def _gru(agg0, agg1, cb, hprev, wih, whh, bih, bhh):
    x = jnp.maximum(agg0[...] + agg1[...] + cb[...], 0.0)
    gi = jnp.dot(x, wih[...], preferred_element_type=_f32) + bih[...]
    gh = jnp.dot(hprev[...], whh[...], preferred_element_type=_f32) + bhh[...]
    r = jax.nn.sigmoid(gi[:, 0:H] + gh[:, 0:H])
    z = jax.nn.sigmoid(gi[:, H:2 * H] + gh[:, H:2 * H])
    ng = jnp.tanh(gi[:, 2 * H:3 * H] + r * gh[:, 2 * H:3 * H])
    return (1.0 - z) * ng + z * hprev[...]


def _step_body(agg0, agg1, cb, hprev, wih, whh, bih, bhh, wbig, h_out, q_out):
    h = _gru(agg0, agg1, cb, hprev, wih, whh, bih, bhh)
    h_out[...] = h
    _store_q(q_out, jnp.dot(h, wbig[...], preferred_element_type=_f32))


def _final_body(agg0, agg1, cb, hprev, wih, whh, bih, bhh, h_out, hg_out):
    h = _gru(agg0, agg1, cb, hprev, wih, whh, bih, bhh)
    h_out[...] = h
    b = pl.program_id(0)
    rows = lax.broadcasted_iota(_i32, (BLK, 1), 0) + b * BLK
    hm = jnp.where(rows < N_NODES, h, 0.0)

    @pl.when(b == 0)
    def _():
        hg_out[...] = jnp.zeros_like(hg_out)

    hg_out[...] += jnp.sum(hm, axis=0, keepdims=True)


_gru_in_specs = [
    pl.BlockSpec((BLK, H), _nblk),
    pl.BlockSpec((BLK, H), _nblk),
    pl.BlockSpec((1, H), _rep),
    pl.BlockSpec((BLK, H), _nblk),
    pl.BlockSpec((H, 3 * H), _rep),
    pl.BlockSpec((H, 3 * H), _rep),
    pl.BlockSpec((1, 3 * H), _rep),
    pl.BlockSpec((1, 3 * H), _rep),
]

_tc_step_call = pl.pallas_call(
    _step_body,
    grid=(N_PAD // BLK,),
    in_specs=_gru_in_specs + [pl.BlockSpec((H, EDGE_VOCAB * H), _rep)],
    out_specs=(
        pl.BlockSpec((BLK, H), _nblk),
        pl.BlockSpec((NG, BLK, 128), lambda b: (0, b, 0)),
    ),
    out_shape=(
        jax.ShapeDtypeStruct((N_PAD, H), _f32),
        jax.ShapeDtypeStruct((NG, N_PAD, 128), _f32),
    ),
)

_tc_final_call = pl.pallas_call(
    _final_body,
    grid=(N_PAD // BLK,),
    in_specs=_gru_in_specs,
    out_specs=(
        pl.BlockSpec((BLK, H), _nblk),
        pl.BlockSpec((1, H), _rep),
    ),
    out_shape=(
        jax.ShapeDtypeStruct((N_PAD, H), _f32),
        jax.ShapeDtypeStruct((1, H), _f32),
    ),
)


def _sc_mesh():
    return plsc.VectorSubcoreMesh(
        core_axis_name="c", subcore_axis_name="s", num_cores=NC, num_subcores=NS
    )


@functools.cache
def _make_sc_step():
    scratch = [
        pltpu.VMEM((NCHUNK, CH), _i32),
        pltpu.VMEM((NCHUNK, CH), _i32),
        pltpu.VMEM((NCHUNK, CH), _i32),
        pltpu.VMEM((NCHUNK, CH), _i32),
        pltpu.VMEM((NCHUNK, CH), _i32),
        pltpu.VMEM((CH, 128), _f32),
        pltpu.VMEM((CH, 128), _f32),
        pltpu.VMEM((CH, H), _f32),
        pltpu.VMEM((CH, H), _f32),
        pltpu.VMEM((CH, H), _f32),
        pltpu.VMEM_SHARED((N_PAD, H), _f32),
        pltpu.SemaphoreType.DMA,
        pltpu.SemaphoreType.DMA,
    ]

    def body(qtab, pkg, sameg, emitg, aggout, pk_v, cidx_v, off_v, same_v,
             emit_v, rows0, rows1, comb0, comb1, zero_v, agg_sh, gs0, gs1):
        cid = lax.axis_index("c")
        sid = lax.axis_index("s")
        wid = cid * NS + sid

        def zrow(j, c):
            zero_v[j, pl.ds(0, 16)] = jnp.zeros((16,), _f32)
            zero_v[j, pl.ds(16, 16)] = jnp.zeros((16,), _f32)
            return c

        lax.fori_loop(0, CH, zrow, 0)
        for j in range(STRIPE // CH):
            pltpu.sync_copy(zero_v, agg_sh.at[pl.ds(sid * STRIPE + j * CH, CH)])

        pltpu.sync_copy(pkg.at[wid], pk_v)
        pltpu.sync_copy(sameg.at[wid], same_v)
        pltpu.sync_copy(emitg.at[wid], emit_v)

        def crow(j, c):
            for k in range(CH // 16):
                sl = pl.ds(k * 16, 16)
                v = pk_v[j, sl]
                cidx_v[j, sl] = jnp.bitwise_and(v, (1 << 18) - 1)
                off_v[j, sl] = jnp.right_shift(v, 18) * H
            return c

        lax.fori_loop(0, NCHUNK, crow, 0)
        plsc.subcore_barrier()

        zvec = jnp.zeros((16,), _f32)

        def fold(j, rows_v, comb_v, acc):
            def fold16(k, acc):
                a0, a1 = acc
                base = k * 16
                sv = same_v[j, pl.ds(base, 16)]
                ov = off_v[j, pl.ds(base, 16)]
                for lane in range(16):
                    o = ov[lane]
                    r0 = rows_v[base + lane, pl.ds(o, 16)]
                    r1 = rows_v[base + lane, pl.ds(o + 16, 16)]
                    cont = sv[lane] != 0
                    a0 = r0 + jnp.where(cont, a0, zvec)
                    a1 = r1 + jnp.where(cont, a1, zvec)
                    comb_v[base + lane, pl.ds(0, 16)] = a0
                    comb_v[base + lane, pl.ds(16, 16)] = a1
                return (a0, a1)

            return lax.fori_loop(0, CH // 16, fold16, acc)

        pltpu.async_copy(qtab.at[cidx_v.at[0]], rows0, gs0)
        pltpu.async_copy(qtab.at[cidx_v.at[1]], rows1, gs1)

        def half(j, rows_v, comb_v, gs, acc):
            pltpu.make_async_copy(qtab.at[cidx_v.at[j]], rows_v, gs).wait()
            acc = fold(j, rows_v, comb_v, acc)
            pltpu.sync_copy(comb_v, agg_sh.at[emit_v.at[j]], add=True)
            jn = jnp.minimum(j + 2, NCHUNK - 1)
            pltpu.async_copy(qtab.at[cidx_v.at[jn]], rows_v, gs)
            return acc

        def chunk2(jj, acc):
            acc = half(2 * jj, rows0, comb0, gs0, acc)
            acc = half(2 * jj + 1, rows1, comb1, gs1, acc)
            return acc

        lax.fori_loop(0, NCHUNK // 2, chunk2, (zvec, zvec))
        pltpu.make_async_copy(qtab.at[cidx_v.at[0]], rows0, gs0).wait()
        pltpu.make_async_copy(qtab.at[cidx_v.at[1]], rows1, gs1).wait()
        plsc.subcore_barrier()

        stripe = pl.ds(sid * STRIPE, STRIPE)
        pltpu.sync_copy(agg_sh.at[stripe], aggout.at[cid, stripe])

    return pl.kernel(
        body,
        out_type=jax.ShapeDtypeStruct((NC, N_PAD, H), _f32),
        mesh=_sc_mesh(),
        scratch_types=scratch,
        compiler_params=pltpu.CompilerParams(use_tc_tiling_on_sc=False),
    )


@functools.cache
def _make_sc_h0():
    GCH = 80

    scratch = [
        pltpu.VMEM((NODE_W,), _i32),
        pltpu.VMEM((GCH, H), _f32),
        pltpu.SemaphoreType.DMA,
    ]

    def body(h0v, nfp, h0out, nf_v, rows_v, sem):
        cid = lax.axis_index("c")
        sid = lax.axis_index("s")
        wid = cid * NS + sid
        base = wid * NODE_W
        pltpu.sync_copy(nfp.at[pl.ds(base, NODE_W)], nf_v)
        for j in range(NODE_W // GCH):
            idx = nf_v.at[pl.ds(j * GCH, GCH)]
            pltpu.async_copy(h0v.at[idx], rows_v, sem).wait()
            pltpu.sync_copy(rows_v, h0out.at[pl.ds(base + j * GCH, GCH)])

    return pl.kernel(
        body,
        out_type=jax.ShapeDtypeStruct((N_PAD, H), _f32),
        mesh=_sc_mesh(),
        scratch_types=scratch,
        compiler_params=pltpu.CompilerParams(use_tc_tiling_on_sc=False),
    )


def kernel(node_emb, edge_emb, proj_W, proj_b, en_W1, en_b1, en_W2, en_b2,
           conv_bias, gru_Wih, gru_Whh, gru_bih, gru_bhh, node_feat,
           edge_feat, edge_index):
    node_feat = node_feat.astype(_i32)
    edge_feat = edge_feat.astype(_i32)
    edge_index = edge_index.astype(_i32)

    cidx_all = ((edge_index[0] + N_PAD * (edge_feat // 4))
                + ((edge_feat % 4) << 18))
    order = jnp.argsort(edge_index[1])
    sdst = edge_index[1][order]
    scidx = cidx_all[order]
    epad = E_PAD - N_EDGES
    cidx_p = jnp.pad(scidx, (0, epad)).reshape(NW, NCHUNK, CH)
    sdst = jnp.pad(sdst, (0, epad),
                   constant_values=DUMP_ROW).reshape(NW, EW)
    prev = jnp.concatenate(
        [jnp.full((NW, 1), -1, _i32), sdst[:, :-1]], axis=1)
    nxt = jnp.concatenate(
        [sdst[:, 1:], jnp.full((NW, 1), -1, _i32)], axis=1)
    same_p = (sdst == prev).astype(_i32).reshape(NW, NCHUNK, CH)
    emit_p = jnp.where(sdst == nxt, DUMP_ROW, sdst).reshape(NW, NCHUNK, CH)
    nf_p = jnp.pad(node_feat, (0, N_PAD - N_NODES))

    pb = proj_b.reshape(1, H)
    b1 = en_b1.reshape(1, EDGE_HIDDEN)
    b2 = en_b2.reshape(1, H * H)
    cb = conv_bias.reshape(1, H)
    wih = gru_Wih.T
    whh = gru_Whh.T
    bih = gru_bih.reshape(1, 3 * H)
    bhh = gru_bhh.reshape(1, 3 * H)

    h0v, ewv = _vocab_call(node_emb, proj_W, pb, edge_emb, en_W1, b1, en_W2, b2)
    wbig = ewv.reshape(EDGE_VOCAB, H, H).transpose(1, 0, 2).reshape(
        H, EDGE_VOCAB * H)
    h = _make_sc_h0()(h0v, nf_p)
    q = _q_call(h, wbig)
    sc_step = _make_sc_step()
    for s in range(N_STEPS):
        agg = sc_step(q.reshape(NG * N_PAD, 128), cidx_p, same_p, emit_p)
        if s < N_STEPS - 1:
            h, q = _tc_step_call(agg[0], agg[1], cb, h, wih, whh, bih, bhh,
                                 wbig)
    h6, hg = _tc_final_call(agg[0], agg[1], cb, h, wih, whh, bih, bhh)
    return (hg, h6[:N_NODES])


# speedup vs baseline: 1.2668x; 1.2668x over previous
import functools

import jax
import jax.numpy as jnp
from jax import lax
from jax.experimental import pallas as pl
from jax.experimental.pallas import tpu as pltpu
from jax.experimental.pallas import tpu_sc as plsc

N_NODES = 10000
N_EDGES = 160000
NODE_VOCAB = 512
EDGE_VOCAB = 64
D_NODE_EMB = 16
D_EDGE_EMB = 32
H = 32
EDGE_HIDDEN = 128
N_STEPS = 6

NC = 2
NS = 16
NW = NC * NS
CH = 128
NCHUNK = 40
E_PAD = NW * NCHUNK * CH
EW = NCHUNK * CH
N_PAD = 10240
NODE_W = N_PAD // NW
STRIPE = N_PAD // NS
DUMP_ROW = 10200
BLK = 512

_f32 = jnp.float32
_i32 = jnp.int32


def _vocab_body(nemb, pw, pb, eemb, w1, b1, w2, b2, h0v, ewv):
    h0 = jnp.dot(nemb[...], pw[...], preferred_element_type=_f32) + pb[...]
    h0v[...] = jnp.maximum(h0, 0.0)
    z = jnp.dot(eemb[...], w1[...], preferred_element_type=_f32) + b1[...]
    z = jnp.maximum(z, 0.0)
    ewv[...] = jnp.dot(z, w2[...], preferred_element_type=_f32) + b2[...]


_vocab_call = pl.pallas_call(
    _vocab_body,
    out_shape=(
        jax.ShapeDtypeStruct((NODE_VOCAB, H), _f32),
        jax.ShapeDtypeStruct((EDGE_VOCAB, H * H), _f32),
    ),
)


def _nblk(b):
    return (b, 0)


def _rep(b):
    return (0, 0)


def _mm_body(a, b, o):
    o[...] = jnp.dot(a[...], b[...], preferred_element_type=_f32)


_q_call = pl.pallas_call(
    _mm_body,
    grid=(N_PAD // BLK,),
    in_specs=[
        pl.BlockSpec((BLK, H), _nblk),
        pl.BlockSpec((H, EDGE_VOCAB * H), _rep),
    ],
    out_specs=pl.BlockSpec((BLK, EDGE_VOCAB * H), _nblk),
    out_shape=jax.ShapeDtypeStruct((N_PAD, EDGE_VOCAB * H), _f32),
)


def _gru(agg0, agg1, cb, hprev, wih, whh, bih, bhh):
    x = jnp.maximum(agg0[...] + agg1[...] + cb[...], 0.0)
    gi = jnp.dot(x, wih[...], preferred_element_type=_f32) + bih[...]
    gh = jnp.dot(hprev[...], whh[...], preferred_element_type=_f32) + bhh[...]
    r = jax.nn.sigmoid(gi[:, 0:H] + gh[:, 0:H])
    z = jax.nn.sigmoid(gi[:, H:2 * H] + gh[:, H:2 * H])
    ng = jnp.tanh(gi[:, 2 * H:3 * H] + r * gh[:, 2 * H:3 * H])
    return (1.0 - z) * ng + z * hprev[...]


def _step_body(agg0, agg1, cb, hprev, wih, whh, bih, bhh, wbig, h_out, q_out):
    h = _gru(agg0, agg1, cb, hprev, wih, whh, bih, bhh)
    h_out[...] = h
    q_out[...] = jnp.dot(h, wbig[...], preferred_element_type=_f32)


def _final_body(agg0, agg1, cb, hprev, wih, whh, bih, bhh, h_out, hg_out):
    h = _gru(agg0, agg1, cb, hprev, wih, whh, bih, bhh)
    h_out[...] = h
    b = pl.program_id(0)
    rows = lax.broadcasted_iota(_i32, (BLK, 1), 0) + b * BLK
    hm = jnp.where(rows < N_NODES, h, 0.0)

    @pl.when(b == 0)
    def _():
        hg_out[...] = jnp.zeros_like(hg_out)

    hg_out[...] += jnp.sum(hm, axis=0, keepdims=True)


_gru_in_specs = [
    pl.BlockSpec((BLK, H), _nblk),
    pl.BlockSpec((BLK, H), _nblk),
    pl.BlockSpec((1, H), _rep),
    pl.BlockSpec((BLK, H), _nblk),
    pl.BlockSpec((H, 3 * H), _rep),
    pl.BlockSpec((H, 3 * H), _rep),
    pl.BlockSpec((1, 3 * H), _rep),
    pl.BlockSpec((1, 3 * H), _rep),
]

_tc_step_call = pl.pallas_call(
    _step_body,
    grid=(N_PAD // BLK,),
    in_specs=_gru_in_specs + [pl.BlockSpec((H, EDGE_VOCAB * H), _rep)],
    out_specs=(
        pl.BlockSpec((BLK, H), _nblk),
        pl.BlockSpec((BLK, EDGE_VOCAB * H), _nblk),
    ),
    out_shape=(
        jax.ShapeDtypeStruct((N_PAD, H), _f32),
        jax.ShapeDtypeStruct((N_PAD, EDGE_VOCAB * H), _f32),
    ),
)

_tc_final_call = pl.pallas_call(
    _final_body,
    grid=(N_PAD // BLK,),
    in_specs=_gru_in_specs,
    out_specs=(
        pl.BlockSpec((BLK, H), _nblk),
        pl.BlockSpec((1, H), _rep),
    ),
    out_shape=(
        jax.ShapeDtypeStruct((N_PAD, H), _f32),
        jax.ShapeDtypeStruct((1, H), _f32),
    ),
)


def _sc_mesh():
    return plsc.VectorSubcoreMesh(
        core_axis_name="c", subcore_axis_name="s", num_cores=NC, num_subcores=NS
    )


@functools.cache
def _make_sc_step():
    scratch = [
        pltpu.VMEM((NCHUNK, CH), _i32),
        pltpu.VMEM((NCHUNK, CH), _i32),
        pltpu.VMEM((NCHUNK, CH), _i32),
        pltpu.VMEM((CH, H), _f32),
        pltpu.VMEM((CH, H), _f32),
        pltpu.VMEM((CH, H), _f32),
        pltpu.VMEM((CH, H), _f32),
        pltpu.VMEM((CH, H), _f32),
        pltpu.VMEM_SHARED((N_PAD, H), _f32),
        pltpu.SemaphoreType.DMA,
        pltpu.SemaphoreType.DMA,
    ]

    def body(qtab, cidxg, sameg, emitg, aggout, cidx_v, same_v,
             emit_v, rows0, rows1, comb0, comb1, zero_v, agg_sh, gs0, gs1):
        cid = lax.axis_index("c")
        sid = lax.axis_index("s")
        wid = cid * NS + sid

        def zrow(j, c):
            zero_v[j, pl.ds(0, 16)] = jnp.zeros((16,), _f32)
            zero_v[j, pl.ds(16, 16)] = jnp.zeros((16,), _f32)
            return c

        lax.fori_loop(0, CH, zrow, 0)
        for j in range(STRIPE // CH):
            pltpu.sync_copy(zero_v, agg_sh.at[pl.ds(sid * STRIPE + j * CH, CH)])

        pltpu.sync_copy(cidxg.at[wid], cidx_v)
        pltpu.sync_copy(sameg.at[wid], same_v)
        pltpu.sync_copy(emitg.at[wid], emit_v)
        plsc.subcore_barrier()

        zvec = jnp.zeros((16,), _f32)

        def fold(j, rows_v, comb_v, acc):
            def fold16(k, acc):
                a0, a1 = acc
                base = k * 16
                sv = same_v[j, pl.ds(base, 16)]
                for lane in range(16):
                    r0 = rows_v[base + lane, pl.ds(0, 16)]
                    r1 = rows_v[base + lane, pl.ds(16, 16)]
                    cont = sv[lane] != 0
                    a0 = r0 + jnp.where(cont, a0, zvec)
                    a1 = r1 + jnp.where(cont, a1, zvec)
                    comb_v[base + lane, pl.ds(0, 16)] = a0
                    comb_v[base + lane, pl.ds(16, 16)] = a1
                return (a0, a1)

            return lax.fori_loop(0, CH // 16, fold16, acc)

        pltpu.async_copy(qtab.at[cidx_v.at[0]], rows0, gs0)
        pltpu.async_copy(qtab.at[cidx_v.at[1]], rows1, gs1)

        def half(j, rows_v, comb_v, gs, acc):
            pltpu.make_async_copy(qtab.at[cidx_v.at[j]], rows_v, gs).wait()
            acc = fold(j, rows_v, comb_v, acc)
            pltpu.sync_copy(comb_v, agg_sh.at[emit_v.at[j]], add=True)
            jn = jnp.minimum(j + 2, NCHUNK - 1)
            pltpu.async_copy(qtab.at[cidx_v.at[jn]], rows_v, gs)
            return acc

        def chunk2(jj, acc):
            acc = half(2 * jj, rows0, comb0, gs0, acc)
            acc = half(2 * jj + 1, rows1, comb1, gs1, acc)
            return acc

        lax.fori_loop(0, NCHUNK // 2, chunk2, (zvec, zvec))
        pltpu.make_async_copy(qtab.at[cidx_v.at[0]], rows0, gs0).wait()
        pltpu.make_async_copy(qtab.at[cidx_v.at[1]], rows1, gs1).wait()
        plsc.subcore_barrier()

        stripe = pl.ds(sid * STRIPE, STRIPE)
        pltpu.sync_copy(agg_sh.at[stripe], aggout.at[cid, stripe])

    return pl.kernel(
        body,
        out_type=jax.ShapeDtypeStruct((NC, N_PAD, H), _f32),
        mesh=_sc_mesh(),
        scratch_types=scratch,
        compiler_params=pltpu.CompilerParams(use_tc_tiling_on_sc=False),
    )


@functools.cache
def _make_sc_h0():
    GCH = 80

    scratch = [
        pltpu.VMEM((NODE_W,), _i32),
        pltpu.VMEM((GCH, H), _f32),
        pltpu.SemaphoreType.DMA,
    ]

    def body(h0v, nfp, h0out, nf_v, rows_v, sem):
        cid = lax.axis_index("c")
        sid = lax.axis_index("s")
        wid = cid * NS + sid
        base = wid * NODE_W
        pltpu.sync_copy(nfp.at[pl.ds(base, NODE_W)], nf_v)
        for j in range(NODE_W // GCH):
            idx = nf_v.at[pl.ds(j * GCH, GCH)]
            pltpu.async_copy(h0v.at[idx], rows_v, sem).wait()
            pltpu.sync_copy(rows_v, h0out.at[pl.ds(base + j * GCH, GCH)])

    return pl.kernel(
        body,
        out_type=jax.ShapeDtypeStruct((N_PAD, H), _f32),
        mesh=_sc_mesh(),
        scratch_types=scratch,
        compiler_params=pltpu.CompilerParams(use_tc_tiling_on_sc=False),
    )


def kernel(node_emb, edge_emb, proj_W, proj_b, en_W1, en_b1, en_W2, en_b2,
           conv_bias, gru_Wih, gru_Whh, gru_bih, gru_bhh, node_feat,
           edge_feat, edge_index):
    node_feat = node_feat.astype(_i32)
    edge_feat = edge_feat.astype(_i32)
    edge_index = edge_index.astype(_i32)

    cidx_all = edge_index[0] * EDGE_VOCAB + edge_feat
    order = jnp.argsort(edge_index[1])
    sdst = edge_index[1][order]
    scidx = cidx_all[order]
    epad = E_PAD - N_EDGES
    cidx_p = jnp.pad(scidx, (0, epad)).reshape(NW, NCHUNK, CH)
    sdst = jnp.pad(sdst, (0, epad),
                   constant_values=DUMP_ROW).reshape(NW, EW)
    prev = jnp.concatenate(
        [jnp.full((NW, 1), -1, _i32), sdst[:, :-1]], axis=1)
    nxt = jnp.concatenate(
        [sdst[:, 1:], jnp.full((NW, 1), -1, _i32)], axis=1)
    same_p = (sdst == prev).astype(_i32).reshape(NW, NCHUNK, CH)
    emit_p = jnp.where(sdst == nxt, DUMP_ROW, sdst).reshape(NW, NCHUNK, CH)
    nf_p = jnp.pad(node_feat, (0, N_PAD - N_NODES))

    pb = proj_b.reshape(1, H)
    b1 = en_b1.reshape(1, EDGE_HIDDEN)
    b2 = en_b2.reshape(1, H * H)
    cb = conv_bias.reshape(1, H)
    wih = gru_Wih.T
    whh = gru_Whh.T
    bih = gru_bih.reshape(1, 3 * H)
    bhh = gru_bhh.reshape(1, 3 * H)

    h0v, ewv = _vocab_call(node_emb, proj_W, pb, edge_emb, en_W1, b1, en_W2, b2)
    wbig = ewv.reshape(EDGE_VOCAB, H, H).transpose(1, 0, 2).reshape(
        H, EDGE_VOCAB * H)
    h = _make_sc_h0()(h0v, nf_p)
    q = _q_call(h, wbig)
    sc_step = _make_sc_step()
    for s in range(N_STEPS):
        agg = sc_step(q.reshape(N_PAD * EDGE_VOCAB, H), cidx_p, same_p,
                      emit_p)
        if s < N_STEPS - 1:
            h, q = _tc_step_call(agg[0], agg[1], cb, h, wih, whh, bih, bhh,
                                 wbig)
    h6, hg = _tc_final_call(agg[0], agg[1], cb, h, wih, whh, bih, bhh)
    return (hg, h6[:N_NODES])


# speedup vs baseline: 1.3233x; 1.0446x over previous
import functools

import jax
import jax.numpy as jnp
from jax import lax
from jax.experimental import pallas as pl
from jax.experimental.pallas import tpu as pltpu
from jax.experimental.pallas import tpu_sc as plsc

N_NODES = 10000
N_EDGES = 160000
NODE_VOCAB = 512
EDGE_VOCAB = 64
D_NODE_EMB = 16
D_EDGE_EMB = 32
H = 32
EDGE_HIDDEN = 128
N_STEPS = 6

NC = 2
NS = 16
NW = NC * NS
CH = 128
NCHUNK = 40
E_PAD = NW * NCHUNK * CH
EW = NCHUNK * CH
N_PAD = 10240
NODE_W = N_PAD // NW
STRIPE = N_PAD // NS
DUMP_ROW = 10200
BLK = 512

_f32 = jnp.float32
_i32 = jnp.int32


def _vocab_body(nemb, pw, pb, eemb, w1, b1, w2, b2, h0v, ewv):
    h0 = jnp.dot(nemb[...], pw[...], preferred_element_type=_f32) + pb[...]
    h0v[...] = jnp.maximum(h0, 0.0)
    z = jnp.dot(eemb[...], w1[...], preferred_element_type=_f32) + b1[...]
    z = jnp.maximum(z, 0.0)
    ewv[...] = jnp.dot(z, w2[...], preferred_element_type=_f32) + b2[...]


_vocab_call = pl.pallas_call(
    _vocab_body,
    out_shape=(
        jax.ShapeDtypeStruct((NODE_VOCAB, H), _f32),
        jax.ShapeDtypeStruct((EDGE_VOCAB, H * H), _f32),
    ),
)


def _nblk(b):
    return (b, 0)


def _rep(b):
    return (0, 0)


def _mm_body(a, b, o):
    o[...] = jnp.dot(a[...], b[...], preferred_element_type=_f32)


_q_call = pl.pallas_call(
    _mm_body,
    grid=(N_PAD // BLK,),
    in_specs=[
        pl.BlockSpec((BLK, H), _nblk),
        pl.BlockSpec((H, EDGE_VOCAB * H), _rep),
    ],
    out_specs=pl.BlockSpec((BLK, EDGE_VOCAB * H), _nblk),
    out_shape=jax.ShapeDtypeStruct((N_PAD, EDGE_VOCAB * H), _f32),
)


def _gru(agg0, agg1, cb, hprev, wih, whh, bih, bhh):
    x = jnp.maximum(agg0[...] + agg1[...] + cb[...], 0.0)
    gi = jnp.dot(x, wih[...], preferred_element_type=_f32) + bih[...]
    gh = jnp.dot(hprev[...], whh[...], preferred_element_type=_f32) + bhh[...]
    r = jax.nn.sigmoid(gi[:, 0:H] + gh[:, 0:H])
    z = jax.nn.sigmoid(gi[:, H:2 * H] + gh[:, H:2 * H])
    ng = jnp.tanh(gi[:, 2 * H:3 * H] + r * gh[:, 2 * H:3 * H])
    return (1.0 - z) * ng + z * hprev[...]


def _step_body(agg0, agg1, cb, hprev, wih, whh, bih, bhh, wbig, h_out, q_out):
    h = _gru(agg0, agg1, cb, hprev, wih, whh, bih, bhh)
    h_out[...] = h
    q_out[...] = jnp.dot(h, wbig[...], preferred_element_type=_f32)


def _final_body(agg0, agg1, cb, hprev, wih, whh, bih, bhh, h_out, hg_out):
    h = _gru(agg0, agg1, cb, hprev, wih, whh, bih, bhh)
    h_out[...] = h
    b = pl.program_id(0)
    rows = lax.broadcasted_iota(_i32, (BLK, 1), 0) + b * BLK
    hm = jnp.where(rows < N_NODES, h, 0.0)

    @pl.when(b == 0)
    def _():
        hg_out[...] = jnp.zeros_like(hg_out)

    hg_out[...] += jnp.sum(hm, axis=0, keepdims=True)


_gru_in_specs = [
    pl.BlockSpec((BLK, H), _nblk),
    pl.BlockSpec((BLK, H), _nblk),
    pl.BlockSpec((1, H), _rep),
    pl.BlockSpec((BLK, H), _nblk),
    pl.BlockSpec((H, 3 * H), _rep),
    pl.BlockSpec((H, 3 * H), _rep),
    pl.BlockSpec((1, 3 * H), _rep),
    pl.BlockSpec((1, 3 * H), _rep),
]

_tc_step_call = pl.pallas_call(
    _step_body,
    grid=(N_PAD // BLK,),
    in_specs=_gru_in_specs + [pl.BlockSpec((H, EDGE_VOCAB * H), _rep)],
    out_specs=(
        pl.BlockSpec((BLK, H), _nblk),
        pl.BlockSpec((BLK, EDGE_VOCAB * H), _nblk),
    ),
    out_shape=(
        jax.ShapeDtypeStruct((N_PAD, H), _f32),
        jax.ShapeDtypeStruct((N_PAD, EDGE_VOCAB * H), _f32),
    ),
)

_tc_final_call = pl.pallas_call(
    _final_body,
    grid=(N_PAD // BLK,),
    in_specs=_gru_in_specs,
    out_specs=(
        pl.BlockSpec((BLK, H), _nblk),
        pl.BlockSpec((1, H), _rep),
    ),
    out_shape=(
        jax.ShapeDtypeStruct((N_PAD, H), _f32),
        jax.ShapeDtypeStruct((1, H), _f32),
    ),
)


def _sc_mesh():
    return plsc.VectorSubcoreMesh(
        core_axis_name="c", subcore_axis_name="s", num_cores=NC, num_subcores=NS
    )


@functools.cache
def _make_sc_step():
    scratch = [
        pltpu.VMEM((NCHUNK, CH), _i32),
        pltpu.VMEM((NCHUNK, CH), _i32),
        pltpu.VMEM((NCHUNK, CH), _i32),
        pltpu.VMEM((CH, H), _f32),
        pltpu.VMEM((CH, H), _f32),
        pltpu.VMEM((CH, H), _f32),
        pltpu.VMEM((CH, H), _f32),
        pltpu.VMEM((CH, H), _f32),
        pltpu.VMEM_SHARED((N_PAD, H), _f32),
        pltpu.SemaphoreType.DMA,
        pltpu.SemaphoreType.DMA,
    ]

    def body(qtab, cidxg, sameg, emitg, aggout, cidx_v, same_v,
             emit_v, rows0, rows1, comb0, comb1, zero_v, agg_sh, gs0, gs1):
        cid = lax.axis_index("c")
        sid = lax.axis_index("s")
        wid = cid * NS + sid

        def zrow(j, c):
            zero_v[j, pl.ds(0, 16)] = jnp.zeros((16,), _f32)
            zero_v[j, pl.ds(16, 16)] = jnp.zeros((16,), _f32)
            return c

        lax.fori_loop(0, CH, zrow, 0)
        for j in range(STRIPE // CH):
            pltpu.sync_copy(zero_v, agg_sh.at[pl.ds(sid * STRIPE + j * CH, CH)])

        pltpu.sync_copy(cidxg.at[wid], cidx_v)
        pltpu.sync_copy(sameg.at[wid], same_v)
        pltpu.sync_copy(emitg.at[wid], emit_v)
        plsc.subcore_barrier()

        zvec = jnp.zeros((16,), _f32)

        def fold(j, rows_v, comb_v, acc):
            def fold16(k, acc):
                a0, a1 = acc
                base = k * 16
                sv = same_v[j, pl.ds(base, 16)]
                for lane in range(16):
                    r0 = rows_v[base + lane, pl.ds(0, 16)]
                    r1 = rows_v[base + lane, pl.ds(16, 16)]
                    cont = sv[lane] != 0
                    a0 = r0 + jnp.where(cont, a0, zvec)
                    a1 = r1 + jnp.where(cont, a1, zvec)
                    comb_v[base + lane, pl.ds(0, 16)] = a0
                    comb_v[base + lane, pl.ds(16, 16)] = a1
                return (a0, a1)

            return lax.fori_loop(0, CH // 16, fold16, acc)

        pltpu.async_copy(qtab.at[cidx_v.at[0]], rows0, gs0)
        pltpu.async_copy(qtab.at[cidx_v.at[1]], rows1, gs1)

        def half(j, rows_v, comb_v, gs, acc):
            pltpu.make_async_copy(qtab.at[cidx_v.at[j]], rows_v, gs).wait()
            acc = fold(j, rows_v, comb_v, acc)
            pltpu.sync_copy(comb_v, agg_sh.at[emit_v.at[j]], add=True)
            jn = jnp.minimum(j + 2, NCHUNK - 1)
            pltpu.async_copy(qtab.at[cidx_v.at[jn]], rows_v, gs)
            return acc

        def chunk2(jj, acc):
            acc = half(2 * jj, rows0, comb0, gs0, acc)
            acc = half(2 * jj + 1, rows1, comb1, gs1, acc)
            return acc

        lax.fori_loop(0, NCHUNK // 2, chunk2, (zvec, zvec))
        pltpu.make_async_copy(qtab.at[cidx_v.at[0]], rows0, gs0).wait()
        pltpu.make_async_copy(qtab.at[cidx_v.at[1]], rows1, gs1).wait()
        plsc.subcore_barrier()

        stripe = pl.ds(sid * STRIPE, STRIPE)
        pltpu.sync_copy(agg_sh.at[stripe], aggout.at[cid, stripe])

    return pl.kernel(
        body,
        out_type=jax.ShapeDtypeStruct((NC, N_PAD, H), _f32),
        mesh=_sc_mesh(),
        scratch_types=scratch,
        compiler_params=pltpu.CompilerParams(use_tc_tiling_on_sc=False),
    )


@functools.cache
def _make_sc_h0():
    GCH = 80

    scratch = [
        pltpu.VMEM((NODE_W,), _i32),
        pltpu.VMEM((GCH, H), _f32),
        pltpu.SemaphoreType.DMA,
    ]

    def body(h0v, nfp, h0out, nf_v, rows_v, sem):
        cid = lax.axis_index("c")
        sid = lax.axis_index("s")
        wid = cid * NS + sid
        base = wid * NODE_W
        pltpu.sync_copy(nfp.at[pl.ds(base, NODE_W)], nf_v)
        for j in range(NODE_W // GCH):
            idx = nf_v.at[pl.ds(j * GCH, GCH)]
            pltpu.async_copy(h0v.at[idx], rows_v, sem).wait()
            pltpu.sync_copy(rows_v, h0out.at[pl.ds(base + j * GCH, GCH)])

    return pl.kernel(
        body,
        out_type=jax.ShapeDtypeStruct((N_PAD, H), _f32),
        mesh=_sc_mesh(),
        scratch_types=scratch,
        compiler_params=pltpu.CompilerParams(use_tc_tiling_on_sc=False),
    )


def kernel(node_emb, edge_emb, proj_W, proj_b, en_W1, en_b1, en_W2, en_b2,
           conv_bias, gru_Wih, gru_Whh, gru_bih, gru_bhh, node_feat,
           edge_feat, edge_index):
    node_feat = node_feat.astype(_i32)
    edge_feat = edge_feat.astype(_i32)
    edge_index = edge_index.astype(_i32)

    cidx_all = edge_index[0] * EDGE_VOCAB + edge_feat
    sdst, scidx = lax.sort((edge_index[1], cidx_all), num_keys=1)
    epad = E_PAD - N_EDGES
    cidx_p = jnp.pad(scidx, (0, epad)).reshape(NW, NCHUNK, CH)
    sdst = jnp.pad(sdst, (0, epad),
                   constant_values=DUMP_ROW).reshape(NW, EW)
    prev = jnp.concatenate(
        [jnp.full((NW, 1), -1, _i32), sdst[:, :-1]], axis=1)
    nxt = jnp.concatenate(
        [sdst[:, 1:], jnp.full((NW, 1), -1, _i32)], axis=1)
    same_p = (sdst == prev).astype(_i32).reshape(NW, NCHUNK, CH)
    emit_p = jnp.where(sdst == nxt, DUMP_ROW, sdst).reshape(NW, NCHUNK, CH)
    nf_p = jnp.pad(node_feat, (0, N_PAD - N_NODES))

    pb = proj_b.reshape(1, H)
    b1 = en_b1.reshape(1, EDGE_HIDDEN)
    b2 = en_b2.reshape(1, H * H)
    cb = conv_bias.reshape(1, H)
    wih = gru_Wih.T
    whh = gru_Whh.T
    bih = gru_bih.reshape(1, 3 * H)
    bhh = gru_bhh.reshape(1, 3 * H)

    h0v, ewv = _vocab_call(node_emb, proj_W, pb, edge_emb, en_W1, b1, en_W2, b2)
    wbig = ewv.reshape(EDGE_VOCAB, H, H).transpose(1, 0, 2).reshape(
        H, EDGE_VOCAB * H)
    h = _make_sc_h0()(h0v, nf_p)
    q = _q_call(h, wbig)
    sc_step = _make_sc_step()
    for s in range(N_STEPS):
        agg = sc_step(q.reshape(N_PAD * EDGE_VOCAB, H), cidx_p, same_p,
                      emit_p)
        if s < N_STEPS - 1:
            h, q = _tc_step_call(agg[0], agg[1], cb, h, wih, whh, bih, bhh,
                                 wbig)
    h6, hg = _tc_final_call(agg[0], agg[1], cb, h, wih, whh, bih, bhh)
    return (hg, h6[:N_NODES])
